# Initial kernel scaffold; baseline (speedup 1.0000x reference)
import functools

import jax
import jax.numpy as jnp
from jax import lax
from jax.experimental import pallas as pl
from jax.experimental.pallas import tpu as pltpu
from jax.experimental.pallas import tpu_sc as plsc

N = 10000
E = 320000
D = 128
L = 16
NSC = 2
NTILE = 16
NW = NSC * NTILE
NPAD = 10240
CHUNK = 128
CPT = 79
EPT = CPT * CHUNK
EPAD = NW * EPT
RPT = NPAD // NTILE

_mesh = plsc.VectorSubcoreMesh(core_axis_name="c", subcore_axis_name="s")


@functools.partial(
    pl.kernel,
    out_type=jax.ShapeDtypeStruct((NW, NPAD), jnp.float32),
    mesh=_mesh,
    scratch_types=[
        pltpu.VMEM((EPT,), jnp.int32),
        pltpu.VMEM((NPAD,), jnp.float32),
    ],
)
def _deg_kernel(dst_hbm, out_hbm, dst_v, hist_v):
    c = lax.axis_index("c")
    s = lax.axis_index("s")
    wid = c * NTILE + s
    zeros16 = jnp.zeros((L,), jnp.float32)

    def zbody(i, _):
        hist_v[pl.ds(i * L, L)] = zeros16
        return ()

    lax.fori_loop(0, NPAD // L, zbody, (), unroll=8)
    pltpu.sync_copy(dst_hbm.at[pl.ds(wid * EPT, EPT)], dst_v)
    ones16 = jnp.ones((L,), jnp.float32)

    def body(k, _):
        idx = dst_v[pl.ds(k * L, L)]
        plsc.addupdate_scatter(hist_v, [idx], ones16)
        return ()

    lax.fori_loop(0, EPT // L, body, (), unroll=8)
    pltpu.sync_copy(hist_v, out_hbm.at[wid])


_BM = 1024


@functools.partial(
    pl.pallas_call,
    grid=(NPAD // _BM,),
    in_specs=[
        pl.BlockSpec((_BM, D), lambda i: (i, 0)),
        pl.BlockSpec((D, D), lambda i: (0, 0)),
        pl.BlockSpec((NW, _BM), lambda i: (0, i)),
    ],
    out_specs=pl.BlockSpec((_BM, D), lambda i: (i, 0)),
    out_shape=jax.ShapeDtypeStruct((NPAD, D), jnp.float32),
)
def _xw_kernel(x_ref, w_ref, h_ref, o_ref):
    deg = jnp.sum(h_ref[...], axis=0) + 1.0
    dis = lax.rsqrt(deg)
    xw = jnp.dot(x_ref[...], w_ref[...], preferred_element_type=jnp.float32)
    o_ref[...] = xw * dis[:, None]


@functools.partial(
    pl.kernel,
    out_type=jax.ShapeDtypeStruct((NSC, NPAD, D), jnp.float32),
    mesh=_mesh,
    scratch_types=[
        pltpu.VMEM((CHUNK,), jnp.int32),
        pltpu.VMEM((CHUNK,), jnp.int32),
        pltpu.VMEM((CHUNK, D), jnp.float32),
        pltpu.VMEM_SHARED((NPAD, D), jnp.float32),
        pltpu.SemaphoreType.DMA,
    ],
)
def _edge_kernel(xw_hbm, src_hbm, dst_hbm, out_hbm, src_v, dst_v, rows_v,
                 acc_sh, sem):
    c = lax.axis_index("c")
    s = lax.axis_index("s")
    wid = c * NTILE + s
    zeros16 = jnp.zeros((L,), jnp.float32)

    def zrow(i, _):
        for j in range(D // L):
            rows_v[i, pl.ds(j * L, L)] = zeros16
        return ()

    lax.fori_loop(0, CHUNK, zrow, ())

    def zacc(i, _):
        pltpu.sync_copy(rows_v, acc_sh.at[pl.ds(s * RPT + i * CHUNK, CHUNK)])
        return ()

    lax.fori_loop(0, RPT // CHUNK, zacc, ())
    plsc.subcore_barrier()
    base = wid * EPT

    def body(g, _):
        off = base + g * CHUNK
        pltpu.sync_copy(src_hbm.at[pl.ds(off, CHUNK)], src_v)
        pltpu.sync_copy(dst_hbm.at[pl.ds(off, CHUNK)], dst_v)
        pltpu.async_copy(xw_hbm.at[src_v], rows_v, sem).wait()
        pltpu.sync_copy(rows_v, acc_sh.at[dst_v], add=True)
        return ()

    lax.fori_loop(0, CPT, body, ())
    plsc.subcore_barrier()

    def wout(i, _):
        r0 = s * RPT + i * CHUNK
        pltpu.sync_copy(acc_sh.at[pl.ds(r0, CHUNK)],
                        out_hbm.at[c, pl.ds(r0, CHUNK)])
        return ()

    lax.fori_loop(0, RPT // CHUNK, wout, ())


_BD = 1000


@functools.partial(
    pl.pallas_call,
    grid=(N // _BD,),
    in_specs=[
        pl.BlockSpec((NSC, _BD, D), lambda i: (0, i, 0)),
        pl.BlockSpec((_BD, D), lambda i: (i, 0)),
        pl.BlockSpec((NW, _BD), lambda i: (0, i)),
        pl.BlockSpec((1, D), lambda i: (0, 0)),
        pl.BlockSpec((1, D), lambda i: (0, 0)),
    ],
    out_specs=pl.BlockSpec((_BD, D), lambda i: (i, 0)),
    out_shape=jax.ShapeDtypeStruct((N, D), jnp.float32),
)
def _finish_kernel(acc_ref, xw_ref, h_ref, b_ref, a_ref, o_ref):
    deg = jnp.sum(h_ref[...], axis=0) + 1.0
    dis = lax.rsqrt(deg)
    acc = acc_ref[...]
    t = (acc[0] + acc[1] + xw_ref[...]) * dis[:, None] + b_ref[...]
    o_ref[...] = jnp.where(t >= 0, t, a_ref[...] * t)


def kernel(x, edge_index, W, b, prelu_a):
    src = edge_index[0]
    dst = edge_index[1]
    pad = jnp.full((EPAD - E,), N, dtype=jnp.int32)
    src_p = jnp.concatenate([src, pad])
    dst_p = jnp.concatenate([dst, pad])
    x_p = jnp.zeros((NPAD, D), x.dtype).at[:N].set(x)
    hist = _deg_kernel(dst_p)
    xw_s = _xw_kernel(x_p, W, hist)
    acc = _edge_kernel(xw_s, src_p, dst_p)
    out = _finish_kernel(acc, xw_s, hist, b.reshape(1, D),
                         prelu_a.reshape(1, D))
    return out


# speedup vs baseline: 16.7494x; 16.7494x over previous
import functools

import jax
import jax.numpy as jnp
from jax import lax
from jax.experimental import pallas as pl
from jax.experimental.pallas import tpu as pltpu
from jax.experimental.pallas import tpu_sc as plsc

N = 10000
E = 320000
D = 128
L = 16
NSC = 2
NTILE = 16
NW = NSC * NTILE
NPAD = 10240
CHUNK = 128
CPT = 79
EPT = CPT * CHUNK
EPAD = NW * EPT
RPT = NPAD // NTILE

_mesh = plsc.VectorSubcoreMesh(core_axis_name="c", subcore_axis_name="s",
                               num_cores=NSC, num_subcores=NTILE)


@functools.partial(
    pl.kernel,
    out_type=jax.ShapeDtypeStruct((NW, NPAD), jnp.float32),
    mesh=_mesh,
    scratch_types=[
        pltpu.VMEM((EPT,), jnp.int32),
        pltpu.VMEM((NPAD,), jnp.float32),
    ],
    compiler_params=pltpu.CompilerParams(needs_layout_passes=False),
)
def _deg_kernel(dst_hbm, out_hbm, dst_v, hist_v):
    c = lax.axis_index("c")
    s = lax.axis_index("s")
    wid = c * NTILE + s
    zeros16 = jnp.zeros((L,), jnp.float32)

    def zbody(i, _):
        hist_v[pl.ds(i * L, L)] = zeros16
        return ()

    lax.fori_loop(0, NPAD // L, zbody, (), unroll=8)
    pltpu.sync_copy(dst_hbm.at[pl.ds(wid * EPT, EPT)], dst_v)
    ones16 = jnp.ones((L,), jnp.float32)

    def body(k, _):
        idx = dst_v[pl.ds(k * L, L)]
        plsc.addupdate_scatter(hist_v, [idx], ones16)
        return ()

    lax.fori_loop(0, EPT // L, body, (), unroll=8)
    pltpu.sync_copy(hist_v, out_hbm.at[wid])


_BM = 1024


@functools.partial(
    pl.pallas_call,
    grid=(NPAD // _BM,),
    in_specs=[
        pl.BlockSpec((_BM, D), lambda i: (i, 0)),
        pl.BlockSpec((D, D), lambda i: (0, 0)),
        pl.BlockSpec((NW, _BM), lambda i: (0, i)),
    ],
    out_specs=pl.BlockSpec((_BM, D), lambda i: (i, 0)),
    out_shape=jax.ShapeDtypeStruct((NPAD, D), jnp.float32),
)
def _xw_kernel(x_ref, w_ref, h_ref, o_ref):
    deg = jnp.sum(h_ref[...], axis=0) + 1.0
    dis = lax.rsqrt(deg)
    xw = jnp.dot(x_ref[...], w_ref[...], preferred_element_type=jnp.float32)
    o_ref[...] = xw * dis[:, None]


@functools.partial(
    pl.kernel,
    out_type=jax.ShapeDtypeStruct((NSC, NPAD, D), jnp.float32),
    mesh=_mesh,
    scratch_types=[
        pltpu.VMEM((CHUNK,), jnp.int32),
        pltpu.VMEM((CHUNK,), jnp.int32),
        pltpu.VMEM((CHUNK, D), jnp.float32),
        pltpu.VMEM_SHARED((NPAD, D), jnp.float32),
        pltpu.SemaphoreType.DMA,
    ],
)
def _edge_kernel(xw_hbm, src_hbm, dst_hbm, out_hbm, src_v, dst_v, rows_v,
                 acc_sh, sem):
    c = lax.axis_index("c")
    s = lax.axis_index("s")
    wid = c * NTILE + s
    zeros16 = jnp.zeros((L,), jnp.float32)

    def zrow(i, _):
        for j in range(D // L):
            rows_v[i, pl.ds(j * L, L)] = zeros16
        return ()

    lax.fori_loop(0, CHUNK, zrow, ())

    def zacc(i, _):
        pltpu.sync_copy(rows_v, acc_sh.at[pl.ds(s * RPT + i * CHUNK, CHUNK)])
        return ()

    lax.fori_loop(0, RPT // CHUNK, zacc, ())
    plsc.subcore_barrier()
    base = wid * EPT

    def body(g, _):
        off = base + g * CHUNK
        pltpu.sync_copy(src_hbm.at[pl.ds(off, CHUNK)], src_v)
        pltpu.sync_copy(dst_hbm.at[pl.ds(off, CHUNK)], dst_v)
        pltpu.async_copy(xw_hbm.at[src_v], rows_v, sem).wait()
        pltpu.sync_copy(rows_v, acc_sh.at[dst_v], add=True)
        return ()

    lax.fori_loop(0, CPT, body, ())
    plsc.subcore_barrier()

    def wout(i, _):
        r0 = s * RPT + i * CHUNK
        pltpu.sync_copy(acc_sh.at[pl.ds(r0, CHUNK)],
                        out_hbm.at[c, pl.ds(r0, CHUNK)])
        return ()

    lax.fori_loop(0, RPT // CHUNK, wout, ())


_BD = 1024


@functools.partial(
    pl.pallas_call,
    grid=(pl.cdiv(N, _BD),),
    in_specs=[
        pl.BlockSpec((NSC, _BD, D), lambda i: (0, i, 0)),
        pl.BlockSpec((_BD, D), lambda i: (i, 0)),
        pl.BlockSpec((NW, _BD), lambda i: (0, i)),
        pl.BlockSpec((1, D), lambda i: (0, 0)),
        pl.BlockSpec((1, D), lambda i: (0, 0)),
    ],
    out_specs=pl.BlockSpec((_BD, D), lambda i: (i, 0)),
    out_shape=jax.ShapeDtypeStruct((N, D), jnp.float32),
)
def _finish_kernel(acc_ref, xw_ref, h_ref, b_ref, a_ref, o_ref):
    deg = jnp.sum(h_ref[...], axis=0) + 1.0
    dis = lax.rsqrt(deg)
    acc = acc_ref[...]
    t = (acc[0] + acc[1] + xw_ref[...]) * dis[:, None] + b_ref[...]
    o_ref[...] = jnp.where(t >= 0, t, a_ref[...] * t)


def kernel(x, edge_index, W, b, prelu_a):
    src = edge_index[0]
    dst = edge_index[1]
    pad = jnp.full((EPAD - E,), N, dtype=jnp.int32)
    src_p = jnp.concatenate([src, pad])
    dst_p = jnp.concatenate([dst, pad])
    x_p = jnp.zeros((NPAD, D), x.dtype).at[:N].set(x)
    hist = _deg_kernel(dst_p)
    xw_s = _xw_kernel(x_p, W, hist)
    acc = _edge_kernel(xw_s, src_p, dst_p)
    out = _finish_kernel(acc, xw_s, hist, b.reshape(1, D),
                         prelu_a.reshape(1, D))
    return out
